# Initial kernel scaffold; baseline (speedup 1.0000x reference)
import jax
import jax.numpy as jnp
from jax.experimental import pallas as pl


def kernel(sent_x, pos_left, pos_right, y, words_table, pos_table):
    raise NotImplementedError("write your pallas kernel here")


# speedup vs baseline: 2.6210x; 2.6210x over previous
import functools

import jax
import jax.numpy as jnp
from jax import lax
from jax.experimental import pallas as pl
from jax.experimental.pallas import tpu as pltpu
from jax.experimental.pallas import tpu_sc as plsc

VOCAB = 13000
POS_VOCAB = 56
EMB = 50
EMB_PAD = 64
B = 4096
L = 50

NC = 2
NS = 16
NW = NC * NS

CB = 8
PB = B // NW
NCHUNK = PB // CB
SEG = 3 * L


def _emb_body(sent_hbm, left_hbm, right_hbm, words_hbm, pos_hbm, out_hbm,
              sent_v, left_v, right_v, rows_v, sem):
    wid = lax.axis_index("s") * NC + lax.axis_index("c")
    base_b = wid * PB

    def chunk(c, carry):
        b0 = base_b + c * CB
        pltpu.sync_copy(sent_hbm.at[pl.ds(b0, CB)], sent_v)
        pltpu.sync_copy(left_hbm.at[pl.ds(b0, CB)], left_v)
        pltpu.sync_copy(right_hbm.at[pl.ds(b0, CB)], right_v)
        descs = []
        for i in range(CB):
            descs.append(pltpu.async_copy(
                words_hbm.at[sent_v.at[i]],
                rows_v.at[pl.ds(i * SEG, L)], sem))
            descs.append(pltpu.async_copy(
                pos_hbm.at[left_v.at[i]],
                rows_v.at[pl.ds(i * SEG + L, L)], sem))
            descs.append(pltpu.async_copy(
                pos_hbm.at[right_v.at[i]],
                rows_v.at[pl.ds(i * SEG + 2 * L, L)], sem))
        for d in descs:
            d.wait()
        pltpu.sync_copy(rows_v, out_hbm.at[pl.ds(b0 * SEG, CB * SEG)])
        return carry

    lax.fori_loop(0, NCHUNK, chunk, 0)


@jax.jit
def _emb_concat(sent_x, pos_left, pos_right, words_table, pos_table):
    words_p = jnp.pad(words_table, ((0, 0), (0, EMB_PAD - EMB)))
    pos_p = jnp.pad(pos_table, ((0, 0), (0, EMB_PAD - EMB)))
    k = pl.kernel(
        _emb_body,
        out_type=jax.ShapeDtypeStruct((B * SEG, EMB_PAD), jnp.float32),
        mesh=plsc.VectorSubcoreMesh(core_axis_name="c", subcore_axis_name="s"),
        scratch_types=[
            pltpu.VMEM((CB, L), jnp.int32),
            pltpu.VMEM((CB, L), jnp.int32),
            pltpu.VMEM((CB, L), jnp.int32),
            pltpu.VMEM((CB * SEG, EMB_PAD), jnp.float32),
            pltpu.SemaphoreType.DMA,
        ],
        compiler_params=pltpu.CompilerParams(use_tc_tiling_on_sc=False),
    )
    out = k(sent_x, pos_left, pos_right, words_p, pos_p)
    return out[:, :EMB].reshape(B, SEG, EMB)


def kernel(sent_x, pos_left, pos_right, y, words_table, pos_table):
    del y
    return _emb_concat(sent_x, pos_left, pos_right, words_table, pos_table)


# speedup vs baseline: 2.9335x; 1.1192x over previous
import jax
import jax.numpy as jnp
from jax import lax
from jax.experimental import pallas as pl
from jax.experimental.pallas import tpu as pltpu
from jax.experimental.pallas import tpu_sc as plsc

VOCAB = 13000
POS_VOCAB = 56
EMB = 50
EMB_PAD = 64
B = 4096
L = 50

NC = 2
NS = 16
NW = NC * NS

CB = 4
PB = B // NW
NCHUNK = PB // CB
SEG = 3 * L
ROWS = CB * SEG


def _emb_body(sent_hbm, left_hbm, right_hbm, words_hbm, pos_hbm, out_hbm,
              sent_v, left_v, right_v, rows_p, rows_c, sem):
    wid = lax.axis_index("s") * NC + lax.axis_index("c")
    base_b = wid * PB

    lane = lax.iota(jnp.int32, 16)
    tail_r = lane >> 1
    tail_c = EMB - 2 + (lane & 1)
    tail_dst = tail_r * EMB + tail_c

    def chunk(c, carry):
        b0 = base_b + c * CB
        pltpu.sync_copy(sent_hbm.at[pl.ds(b0, CB)], sent_v)
        pltpu.sync_copy(left_hbm.at[pl.ds(b0, CB)], left_v)
        pltpu.sync_copy(right_hbm.at[pl.ds(b0, CB)], right_v)
        descs = []
        for i in range(CB):
            descs.append(pltpu.async_copy(
                words_hbm.at[sent_v.at[i]],
                rows_p.at[pl.ds(i * SEG, L)], sem))
            descs.append(pltpu.async_copy(
                pos_hbm.at[left_v.at[i]],
                rows_p.at[pl.ds(i * SEG + L, L)], sem))
            descs.append(pltpu.async_copy(
                pos_hbm.at[right_v.at[i]],
                rows_p.at[pl.ds(i * SEG + 2 * L, L)], sem))
        for d in descs:
            d.wait()

        def group(g, carry2):
            r0 = g * 8
            dst0 = r0 * EMB
            for i in range(8):
                row = rows_p.at[r0 + i]
                for k in range(3):
                    rows_c[pl.ds(dst0 + i * EMB + 16 * k, 16)] = (
                        row[pl.ds(16 * k, 16)])
            tv = plsc.load_gather(rows_p, [tail_r + r0, tail_c])
            plsc.store_scatter(rows_c, [tail_dst + dst0], tv)
            return carry2

        lax.fori_loop(0, ROWS // 8, group, 0)
        pltpu.sync_copy(rows_c, out_hbm.at[pl.ds(b0 * SEG * EMB, ROWS * EMB)])
        return carry

    lax.fori_loop(0, NCHUNK, chunk, 0)


@jax.jit
def _emb_concat(sent_x, pos_left, pos_right, words_table, pos_table):
    words_p = jnp.pad(words_table, ((0, 0), (0, EMB_PAD - EMB)))
    pos_p = jnp.pad(pos_table, ((0, 0), (0, EMB_PAD - EMB)))
    k = pl.kernel(
        _emb_body,
        out_type=jax.ShapeDtypeStruct((B * SEG * EMB,), jnp.float32),
        mesh=plsc.VectorSubcoreMesh(core_axis_name="c", subcore_axis_name="s"),
        scratch_types=[
            pltpu.VMEM((CB, L), jnp.int32),
            pltpu.VMEM((CB, L), jnp.int32),
            pltpu.VMEM((CB, L), jnp.int32),
            pltpu.VMEM((ROWS, EMB_PAD), jnp.float32),
            pltpu.VMEM((ROWS * EMB,), jnp.float32),
            pltpu.SemaphoreType.DMA,
        ],
        compiler_params=pltpu.CompilerParams(
            use_tc_tiling_on_sc=False, needs_layout_passes=False),
    )
    out = k(sent_x, pos_left, pos_right, words_p, pos_p)
    return out.reshape(B, SEG, EMB)


def kernel(sent_x, pos_left, pos_right, y, words_table, pos_table):
    del y
    return _emb_concat(sent_x, pos_left, pos_right, words_table, pos_table)
